# baseline (speedup 1.0000x reference)
import functools

import jax
import jax.numpy as jnp
from jax.experimental import pallas as pl
from jax.experimental.pallas import tpu as pltpu

B, P, D = 64, 1024, 768
PB = 256


def kernel(patch_embeddings, pos_table):
    grid = (P // PB, B)

    def body(x_ref, pos_ref, out_ref):
        out_ref[...] = x_ref[...] + pos_ref[...]

    return pl.pallas_call(
        body,
        grid=grid,
        in_specs=[
            pl.BlockSpec((1, PB, D), lambda p, b: (b, p, 0)),
            pl.BlockSpec((PB, D), lambda p, b: (p, 0)),
        ],
        out_specs=pl.BlockSpec((1, PB, D), lambda p, b: (b, p, 0)),
        out_shape=jax.ShapeDtypeStruct((B, P, D), jnp.float32),
        compiler_params=pltpu.CompilerParams(
            dimension_semantics=("arbitrary", "arbitrary"),
        ),
    )(patch_embeddings, pos_table)


# speedup vs baseline: 1.4017x; 1.4017x over previous
import functools

import jax
import jax.numpy as jnp
from jax import lax
from jax.experimental import pallas as pl
from jax.experimental.pallas import tpu as pltpu
from jax.experimental.pallas import tpu_sc as plsc

B, P, D = 64, 1024, 768
NC, NS = 2, 16
NW = NC * NS
LANES = 16
PW = P // NW
SUB = 16
NB = 2
NPAIR = 3
G = (B // NB) * (PW // SUB)


def kernel(patch_embeddings, pos_table):
    mesh = plsc.VectorSubcoreMesh(core_axis_name="c", subcore_axis_name="s")

    @functools.partial(
        pl.kernel,
        out_type=jax.ShapeDtypeStruct((B, P, D), jnp.float32),
        mesh=mesh,
        compiler_params=pltpu.CompilerParams(use_tc_tiling_on_sc=True),
        scratch_types=[
            pltpu.VMEM((PW, D), jnp.float32),
            [pltpu.VMEM((SUB, D), jnp.float32)] * (NB * NPAIR),
            [pltpu.SemaphoreType.DMA] * (NB * NPAIR),
            [pltpu.SemaphoreType.DMA] * (NB * NPAIR),
        ],
    )
    def body(x_hbm, pos_hbm, out_hbm, pos_v, bufs, gsem, ssem):
        c = lax.axis_index("c")
        s = lax.axis_index("s")
        w = s * NC + c
        p0 = w * PW
        pltpu.sync_copy(pos_hbm.at[pl.ds(p0, PW)], pos_v)

        def start_gather(g, e):
            h = g % 2
            b = (g // 2) * NB
            rows = pl.ds(p0 + h * SUB, SUB)
            for j in range(NB):
                pltpu.async_copy(
                    x_hbm.at[b + j, rows], bufs[NB * e + j], gsem[NB * e + j]
                )

        def run_group(g, e):
            h = g % 2
            b = (g // 2) * NB
            rows = pl.ds(p0 + h * SUB, SUB)
            for j in range(NB):
                pltpu.make_async_copy(
                    x_hbm.at[b + j, rows], bufs[NB * e + j], gsem[NB * e + j]
                ).wait()

            bA = bufs[NB * e]
            bB = bufs[NB * e + 1]
            prow = h * SUB

            @plsc.parallel_loop(0, SUB, 1)
            def _(r):
                pr = prow + r

                @plsc.parallel_loop(0, D // LANES, 1, unroll=4)
                def _(i):
                    sl = pl.ds(i * LANES, LANES)
                    pv = pos_v[pr, sl]
                    bA[r, sl] = bA[r, sl] + pv
                    bB[r, sl] = bB[r, sl] + pv

            for j in range(NB):
                pltpu.async_copy(
                    bufs[NB * e + j], out_hbm.at[b + j, rows], ssem[NB * e + j]
                )

        def wait_scatter_pair(e2):
            for j in range(NB):
                pltpu.make_async_copy(
                    bufs[NB * e2 + j],
                    out_hbm.at[0, pl.ds(0, SUB)],
                    ssem[NB * e2 + j],
                ).wait()

        start_gather(0, 0)
        start_gather(1, 1)

        def per_iter(u, _):
            for e in range(NPAIR):
                g = NPAIR * u + e
                e2 = (e + 2) % NPAIR
                run_group(g, e)
                if e == 0:
                    @pl.when(u > 0)
                    def _():
                        wait_scatter_pair(e2)
                    start_gather(g + 2, e2)
                else:
                    wait_scatter_pair(e2)

                    @pl.when(g + 2 < G)
                    def _():
                        start_gather(g + 2, e2)
            return 0

        lax.fori_loop(0, (G - 1) // NPAIR, per_iter, 0)

        run_group(G - 1, 0)
        wait_scatter_pair((0 + 2) % NPAIR)
        wait_scatter_pair(0)

    return body(patch_embeddings, pos_table)


# speedup vs baseline: 1.4066x; 1.0035x over previous
import functools

import jax
import jax.numpy as jnp
from jax import lax
from jax.experimental import pallas as pl
from jax.experimental.pallas import tpu as pltpu
from jax.experimental.pallas import tpu_sc as plsc

B, P, D = 64, 1024, 768
NC, NS = 2, 16
NW = NC * NS
LANES = 16
PW = P // NW
SUB = 16
NB = 2
NPAIR = 3
G = (B // NB) * (PW // SUB)


def kernel(patch_embeddings, pos_table):
    mesh = plsc.VectorSubcoreMesh(core_axis_name="c", subcore_axis_name="s")

    @functools.partial(
        pl.kernel,
        out_type=jax.ShapeDtypeStruct((B, P, D), jnp.float32),
        mesh=mesh,
        compiler_params=pltpu.CompilerParams(
            use_tc_tiling_on_sc=True,
            skip_device_barrier=True,
            disable_bounds_checks=True,
            disable_semaphore_checks=True,
        ),
        scratch_types=[
            pltpu.VMEM((PW, D), jnp.float32),
            [pltpu.VMEM((SUB, D), jnp.float32)] * (NB * NPAIR),
            [pltpu.SemaphoreType.DMA] * (NB * NPAIR),
            [pltpu.SemaphoreType.DMA] * (NB * NPAIR),
        ],
    )
    def body(x_hbm, pos_hbm, out_hbm, pos_v, bufs, gsem, ssem):
        c = lax.axis_index("c")
        s = lax.axis_index("s")
        w = s * NC + c
        p0 = w * PW
        pltpu.sync_copy(pos_hbm.at[pl.ds(p0, PW)], pos_v)

        def start_gather(g, e):
            h = g % 2
            b = (g // 2) * NB
            rows = pl.ds(p0 + h * SUB, SUB)
            for j in range(NB):
                pltpu.async_copy(
                    x_hbm.at[b + j, rows], bufs[NB * e + j], gsem[NB * e + j]
                )

        def run_group(g, e):
            h = g % 2
            b = (g // 2) * NB
            rows = pl.ds(p0 + h * SUB, SUB)
            for j in range(NB):
                pltpu.make_async_copy(
                    x_hbm.at[b + j, rows], bufs[NB * e + j], gsem[NB * e + j]
                ).wait()

            bA = bufs[NB * e]
            bB = bufs[NB * e + 1]
            prow = h * SUB

            @plsc.parallel_loop(0, SUB, 1)
            def _(r):
                pr = prow + r

                @plsc.parallel_loop(0, D // LANES, 1, unroll=4)
                def _(i):
                    sl = pl.ds(i * LANES, LANES)
                    pv = pos_v[pr, sl]
                    bA[r, sl] = bA[r, sl] + pv
                    bB[r, sl] = bB[r, sl] + pv

            for j in range(NB):
                pltpu.async_copy(
                    bufs[NB * e + j], out_hbm.at[b + j, rows], ssem[NB * e + j]
                )

        def wait_scatter_pair(e2):
            for j in range(NB):
                pltpu.make_async_copy(
                    bufs[NB * e2 + j],
                    out_hbm.at[0, pl.ds(0, SUB)],
                    ssem[NB * e2 + j],
                ).wait()

        start_gather(0, 0)
        start_gather(1, 1)

        def per_iter(u, _):
            for e in range(NPAIR):
                g = NPAIR * u + e
                e2 = (e + 2) % NPAIR
                run_group(g, e)
                if e == 0:
                    @pl.when(u > 0)
                    def _():
                        wait_scatter_pair(e2)
                    start_gather(g + 2, e2)
                else:
                    wait_scatter_pair(e2)

                    @pl.when(g + 2 < G)
                    def _():
                        start_gather(g + 2, e2)
            return 0

        lax.fori_loop(0, (G - 1) // NPAIR, per_iter, 0)

        run_group(G - 1, 0)
        wait_scatter_pair((0 + 2) % NPAIR)
        wait_scatter_pair(0)

    return body(patch_embeddings, pos_table)
